# baseline (speedup 1.0000x reference)
import functools

import numpy as np
import jax
import jax.numpy as jnp
from jax import lax
from jax.experimental import pallas as pl
from jax.experimental.pallas import tpu as pltpu
from jax.experimental.pallas import tpu_sc as plsc

_N_BLOCK = 4096
_MAX = 128

_SEL = np.zeros((9, 5 * _MAX), dtype=np.float32)
for _g in range(3):
    _SEL[3 + _g, _g * _MAX:(_g + 1) * _MAX] = 1.0
_SEL[6, 3 * _MAX:4 * _MAX] = 1.0
_SEL[6, 4 * _MAX:5 * _MAX] = -1.0
_SEL[7, 4 * _MAX:5 * _MAX] = 1.0


_SC_CHUNK = 32


def _z_sc_body(bflat_hbm, dist_hbm, z_hbm,
               nears, fars, dists, ztile):
    info = plsc.get_sparse_core_info()
    nw = info.num_cores * info.num_subcores
    n = z_hbm.shape[0]
    rw = n // nw
    wid = lax.axis_index("s") * info.num_cores + lax.axis_index("c")
    base = wid * rw

    pltpu.sync_copy(bflat_hbm.at[pl.ds(base, rw)], nears)
    pltpu.sync_copy(bflat_hbm.at[pl.ds(n + base, rw)], fars)
    pltpu.sync_copy(dist_hbm.at[pl.ds(base, rw)], dists)

    iota16 = lax.iota(jnp.int32, 16)

    def chunk_body(c, carry):
        for g in range(_SC_CHUNK // 16):
            off = c * _SC_CHUNK + g * 16
            n16 = nears[pl.ds(off, 16)]
            f16 = fars[pl.ds(off, 16)]
            d16 = dists[pl.ds(off, 16)]
            fmn16 = f16 - n16
            inv16 = jnp.where(d16 < 25.0, 1.0 / 127.0,
                    jnp.where(d16 < 50.0, 1.0 / 63.0,
                    jnp.where(d16 < 100.0, 1.0 / 31.0, 1.0 / 15.0)))
            nsf16 = jnp.where(d16 < 25.0, 128.0,
                    jnp.where(d16 < 50.0, 64.0,
                    jnp.where(d16 < 100.0, 32.0, 16.0)))
            for r in range(16):
                idx = jnp.full((16,), r, jnp.int32)
                nb = n16.at[idx].get(mode="promise_in_bounds")
                fb = fmn16.at[idx].get(mode="promise_in_bounds")
                ib = inv16.at[idx].get(mode="promise_in_bounds")
                sb = nsf16.at[idx].get(mode="promise_in_bounds")
                row = g * 16 + r
                for b8 in range(8):
                    jf = (iota16 + 16 * b8).astype(jnp.float32)
                    zc = nb + fb * (jf * ib)
                    zm = jnp.where(jf < sb, zc, 0.0)
                    ztile[row, pl.ds(16 * b8, 16)] = zm
        pltpu.sync_copy(ztile, z_hbm.at[pl.ds(base + c * _SC_CHUNK, _SC_CHUNK)])
        return carry

    lax.fori_loop(0, rw // _SC_CHUNK, chunk_body, 0)


def _z_on_sparsecore(bounds_t, dist):
    n = dist.shape[0]
    rw = n // 32
    return pl.kernel(
        _z_sc_body,
        out_type=jax.ShapeDtypeStruct((n, 128), jnp.float32),
        mesh=plsc.VectorSubcoreMesh(core_axis_name="c", subcore_axis_name="s"),
        scratch_types=[
            pltpu.VMEM((rw,), jnp.float32),
            pltpu.VMEM((rw,), jnp.float32),
            pltpu.VMEM((rw,), jnp.float32),
            pltpu.VMEM((_SC_CHUNK, 128), jnp.float32),
        ],
    )(bounds_t.reshape(2 * n), dist)


def _body(featst_ref, sel_ref, w1t_ref, b1_ref, w2t_ref, b2_ref,
          pts_ref, outt_ref):
    ft = featst_ref[...]
    sel = sel_ref[...]
    b = lax.dot_general(ft, sel, (((0,), (0,)), ((), ())),
                        preferred_element_type=jnp.float32)
    near = b[:, 3 * _MAX:4 * _MAX]
    fmn = b[:, 4 * _MAX:5 * _MAX]
    ot = jnp.transpose(ft[0:3, :], (1, 0))
    dcol = jnp.transpose(ft[8:9, :], (1, 0))

    R = ft.shape[1]
    dmin = jnp.min(ft[8:9, :])

    @pl.when(dmin >= 100.0)
    def _fast():
        jr = lax.broadcasted_iota(jnp.int32, (1, _MAX), 1)
        m1 = jnp.where(jr < 16, 1.0, 0.0)
        tc = jnp.where(jr < 16, jr.astype(jnp.float32) * (1.0 / 15.0), 0.0)
        z = near * m1 + fmn * tc
        for c in range(3):
            d_c = b[:, c * _MAX:(c + 1) * _MAX]
            pts_ref[c, :, :] = ot[:, c:c + 1] * m1 + d_c * z

    @pl.when(dmin < 100.0)
    def _general():
        jf = lax.broadcasted_iota(jnp.int32, (R, _MAX), 1).astype(jnp.float32)
        inv = jnp.where(dcol < 25.0, 1.0 / 127.0,
              jnp.where(dcol < 50.0, 1.0 / 63.0,
              jnp.where(dcol < 100.0, 1.0 / 31.0, 1.0 / 15.0)))
        t = jf * inv
        live = t <= 1.003
        z = near + fmn * t
        for c in range(3):
            d_c = b[:, c * _MAX:(c + 1) * _MAX]
            pts_ref[c, :, :] = jnp.where(live, ot[:, c:c + 1] + d_c * z, 0.0)

    h = jnp.maximum(
        jnp.dot(w1t_ref[...], ft, preferred_element_type=jnp.float32) + b1_ref[...],
        0.0)
    outt_ref[...] = jnp.dot(w2t_ref[...], h, preferred_element_type=jnp.float32) + b2_ref[...]


def kernel(rays_o, rays_d, bounds, distances, W1, b1, W2, b2):
    N = rays_o.shape[0]
    feats_t = jnp.concatenate([rays_o.T, rays_d.T, bounds.T, distances[None, :]], axis=0)
    R = _N_BLOCK
    grid = (N // R,)

    z_vals = _z_on_sparsecore(bounds.T, distances)

    pts_t, out_t = pl.pallas_call(
        _body,
        grid=grid,
        in_specs=[
            pl.BlockSpec((9, R), lambda i: (0, i)),
            pl.BlockSpec((9, 5 * _MAX), lambda i: (0, 0)),
            pl.BlockSpec((256, 9), lambda i: (0, 0)),
            pl.BlockSpec((256, 1), lambda i: (0, 0)),
            pl.BlockSpec((4, 256), lambda i: (0, 0)),
            pl.BlockSpec((4, 1), lambda i: (0, 0)),
        ],
        out_specs=[
            pl.BlockSpec((3, R, _MAX), lambda i: (0, i, 0)),
            pl.BlockSpec((4, R), lambda i: (0, i)),
        ],
        out_shape=[
            jax.ShapeDtypeStruct((3, N, _MAX), jnp.float32),
            jax.ShapeDtypeStruct((4, N), jnp.float32),
        ],
    )(feats_t, jnp.asarray(_SEL), W1.T, b1.reshape(256, 1), W2.T, b2.reshape(4, 1))

    return jnp.transpose(pts_t, (1, 2, 0)), z_vals, out_t.T


# speedup vs baseline: 1.0219x; 1.0219x over previous
import functools

import numpy as np
import jax
import jax.numpy as jnp
from jax import lax
from jax.experimental import pallas as pl
from jax.experimental.pallas import tpu as pltpu
from jax.experimental.pallas import tpu_sc as plsc

_N_BLOCK = 2048
_MAX = 128

_SEL = np.zeros((9, 9 * _MAX), dtype=np.float32)
for _g in range(9):
    _SEL[_g, _g * _MAX:(_g + 1) * _MAX] = 1.0
_SEL[6, 7 * _MAX:8 * _MAX] = -1.0


_SC_CHUNK = 32


def _z_sc_body(bflat_hbm, dist_hbm, z_hbm,
               nears, fars, dists, ztile):
    info = plsc.get_sparse_core_info()
    nw = info.num_cores * info.num_subcores
    n = z_hbm.shape[0]
    rw = n // nw
    wid = lax.axis_index("s") * info.num_cores + lax.axis_index("c")
    base = wid * rw

    pltpu.sync_copy(bflat_hbm.at[pl.ds(base, rw)], nears)
    pltpu.sync_copy(bflat_hbm.at[pl.ds(n + base, rw)], fars)
    pltpu.sync_copy(dist_hbm.at[pl.ds(base, rw)], dists)

    iota16 = lax.iota(jnp.int32, 16)

    def chunk_body(c, carry):
        for g in range(_SC_CHUNK // 16):
            off = c * _SC_CHUNK + g * 16
            n16 = nears[pl.ds(off, 16)]
            f16 = fars[pl.ds(off, 16)]
            d16 = dists[pl.ds(off, 16)]
            fmn16 = f16 - n16
            inv16 = jnp.where(d16 < 25.0, 1.0 / 127.0,
                    jnp.where(d16 < 50.0, 1.0 / 63.0,
                    jnp.where(d16 < 100.0, 1.0 / 31.0, 1.0 / 15.0)))
            nsf16 = jnp.where(d16 < 25.0, 128.0,
                    jnp.where(d16 < 50.0, 64.0,
                    jnp.where(d16 < 100.0, 32.0, 16.0)))
            for r in range(16):
                idx = jnp.full((16,), r, jnp.int32)
                nb = n16.at[idx].get(mode="promise_in_bounds")
                fb = fmn16.at[idx].get(mode="promise_in_bounds")
                ib = inv16.at[idx].get(mode="promise_in_bounds")
                sb = nsf16.at[idx].get(mode="promise_in_bounds")
                row = g * 16 + r
                for b8 in range(8):
                    jf = (iota16 + 16 * b8).astype(jnp.float32)
                    zc = nb + fb * (jf * ib)
                    zm = jnp.where(jf < sb, zc, 0.0)
                    ztile[row, pl.ds(16 * b8, 16)] = zm
        pltpu.sync_copy(ztile, z_hbm.at[pl.ds(base + c * _SC_CHUNK, _SC_CHUNK)])
        return carry

    lax.fori_loop(0, rw // _SC_CHUNK, chunk_body, 0)


def _z_on_sparsecore(bounds_t, dist):
    n = dist.shape[0]
    rw = n // 32
    return pl.kernel(
        _z_sc_body,
        out_type=jax.ShapeDtypeStruct((n, 128), jnp.float32),
        mesh=plsc.VectorSubcoreMesh(core_axis_name="c", subcore_axis_name="s"),
        scratch_types=[
            pltpu.VMEM((rw,), jnp.float32),
            pltpu.VMEM((rw,), jnp.float32),
            pltpu.VMEM((rw,), jnp.float32),
            pltpu.VMEM((_SC_CHUNK, 128), jnp.float32),
        ],
    )(bounds_t.reshape(2 * n), dist)


def _body(featst_ref, sel_ref, w1t_ref, b1_ref, w2t_ref, b2_ref,
          pts_ref, outt_ref):
    ft = featst_ref[...]
    sel = sel_ref[...]
    b = lax.dot_general(ft, sel, (((0,), (0,)), ((), ())),
                        preferred_element_type=jnp.float32)
    near = b[:, 6 * _MAX:7 * _MAX]
    fmn = b[:, 7 * _MAX:8 * _MAX]
    dist = b[:, 8 * _MAX:9 * _MAX]

    R = ft.shape[1]
    jf = lax.broadcasted_iota(jnp.int32, (R, _MAX), 1).astype(jnp.float32)

    inv = jnp.where(dist < 25.0, 1.0 / 127.0,
          jnp.where(dist < 50.0, 1.0 / 63.0,
          jnp.where(dist < 100.0, 1.0 / 31.0, 1.0 / 15.0)))

    t = jf * inv
    live = t <= 1.003
    z = near + fmn * t

    for c in range(3):
        o_c = b[:, c * _MAX:(c + 1) * _MAX]
        d_c = b[:, (c + 3) * _MAX:(c + 4) * _MAX]
        pts_ref[c, :, :] = jnp.where(live, o_c + d_c * z, 0.0)

    h = jnp.maximum(
        jnp.dot(w1t_ref[...], ft, preferred_element_type=jnp.float32) + b1_ref[...],
        0.0)
    outt_ref[...] = jnp.dot(w2t_ref[...], h, preferred_element_type=jnp.float32) + b2_ref[...]


def kernel(rays_o, rays_d, bounds, distances, W1, b1, W2, b2):
    N = rays_o.shape[0]
    feats_t = jnp.concatenate([rays_o.T, rays_d.T, bounds.T, distances[None, :]], axis=0)
    R = _N_BLOCK
    grid = (N // R,)

    z_vals = _z_on_sparsecore(bounds.T, distances)

    pts_t, out_t = pl.pallas_call(
        _body,
        grid=grid,
        in_specs=[
            pl.BlockSpec((9, R), lambda i: (0, i)),
            pl.BlockSpec((9, 9 * _MAX), lambda i: (0, 0)),
            pl.BlockSpec((256, 9), lambda i: (0, 0)),
            pl.BlockSpec((256, 1), lambda i: (0, 0)),
            pl.BlockSpec((4, 256), lambda i: (0, 0)),
            pl.BlockSpec((4, 1), lambda i: (0, 0)),
        ],
        out_specs=[
            pl.BlockSpec((3, R, _MAX), lambda i: (0, i, 0)),
            pl.BlockSpec((4, R), lambda i: (0, i)),
        ],
        out_shape=[
            jax.ShapeDtypeStruct((3, N, _MAX), jnp.float32),
            jax.ShapeDtypeStruct((4, N), jnp.float32),
        ],
    )(feats_t, jnp.asarray(_SEL), W1.T, b1.reshape(256, 1), W2.T, b2.reshape(4, 1))

    return jnp.transpose(pts_t, (1, 2, 0)), z_vals, out_t.T


# speedup vs baseline: 1.0578x; 1.0352x over previous
import functools

import numpy as np
import jax
import jax.numpy as jnp
from jax import lax
from jax.experimental import pallas as pl
from jax.experimental.pallas import tpu as pltpu
from jax.experimental.pallas import tpu_sc as plsc

_N_BLOCK = 4096
_MAX = 128

_SEL = np.zeros((9, 9 * _MAX), dtype=np.float32)
for _g in range(9):
    _SEL[_g, _g * _MAX:(_g + 1) * _MAX] = 1.0
_SEL[6, 7 * _MAX:8 * _MAX] = -1.0


_SC_CHUNK = 32


def _z_sc_body(bflat_hbm, dist_hbm, z_hbm,
               nears, fars, dists, ztile):
    info = plsc.get_sparse_core_info()
    nw = info.num_cores * info.num_subcores
    n = z_hbm.shape[0]
    rw = n // nw
    wid = lax.axis_index("s") * info.num_cores + lax.axis_index("c")
    base = wid * rw

    pltpu.sync_copy(bflat_hbm.at[pl.ds(base, rw)], nears)
    pltpu.sync_copy(bflat_hbm.at[pl.ds(n + base, rw)], fars)
    pltpu.sync_copy(dist_hbm.at[pl.ds(base, rw)], dists)

    iota16 = lax.iota(jnp.int32, 16)

    def chunk_body(c, carry):
        for g in range(_SC_CHUNK // 16):
            off = c * _SC_CHUNK + g * 16
            n16 = nears[pl.ds(off, 16)]
            f16 = fars[pl.ds(off, 16)]
            d16 = dists[pl.ds(off, 16)]
            fmn16 = f16 - n16
            inv16 = jnp.where(d16 < 25.0, 1.0 / 127.0,
                    jnp.where(d16 < 50.0, 1.0 / 63.0,
                    jnp.where(d16 < 100.0, 1.0 / 31.0, 1.0 / 15.0)))
            nsf16 = jnp.where(d16 < 25.0, 128.0,
                    jnp.where(d16 < 50.0, 64.0,
                    jnp.where(d16 < 100.0, 32.0, 16.0)))
            for r in range(16):
                idx = jnp.full((16,), r, jnp.int32)
                nb = n16.at[idx].get(mode="promise_in_bounds")
                fb = fmn16.at[idx].get(mode="promise_in_bounds")
                ib = inv16.at[idx].get(mode="promise_in_bounds")
                sb = nsf16.at[idx].get(mode="promise_in_bounds")
                row = g * 16 + r
                for b8 in range(8):
                    jf = (iota16 + 16 * b8).astype(jnp.float32)
                    zc = nb + fb * (jf * ib)
                    zm = jnp.where(jf < sb, zc, 0.0)
                    ztile[row, pl.ds(16 * b8, 16)] = zm
        pltpu.sync_copy(ztile, z_hbm.at[pl.ds(base + c * _SC_CHUNK, _SC_CHUNK)])
        return carry

    lax.fori_loop(0, rw // _SC_CHUNK, chunk_body, 0)


def _z_on_sparsecore(bounds_t, dist):
    n = dist.shape[0]
    rw = n // 32
    return pl.kernel(
        _z_sc_body,
        out_type=jax.ShapeDtypeStruct((n, 128), jnp.float32),
        mesh=plsc.VectorSubcoreMesh(core_axis_name="c", subcore_axis_name="s"),
        scratch_types=[
            pltpu.VMEM((rw,), jnp.float32),
            pltpu.VMEM((rw,), jnp.float32),
            pltpu.VMEM((rw,), jnp.float32),
            pltpu.VMEM((_SC_CHUNK, 128), jnp.float32),
        ],
    )(bounds_t.reshape(2 * n), dist)


def _body(featst_ref, sel_ref, w1t_ref, b1_ref, w2t_ref, b2_ref,
          pts_ref, outt_ref):
    ft = featst_ref[...]
    sel = sel_ref[...]
    b = lax.dot_general(ft, sel, (((0,), (0,)), ((), ())),
                        preferred_element_type=jnp.float32)
    near = b[:, 6 * _MAX:7 * _MAX]
    fmn = b[:, 7 * _MAX:8 * _MAX]
    dist = b[:, 8 * _MAX:9 * _MAX]

    R = ft.shape[1]
    jf = lax.broadcasted_iota(jnp.int32, (R, _MAX), 1).astype(jnp.float32)

    inv = jnp.where(dist < 25.0, 1.0 / 127.0,
          jnp.where(dist < 50.0, 1.0 / 63.0,
          jnp.where(dist < 100.0, 1.0 / 31.0, 1.0 / 15.0)))

    t = jf * inv
    live = t <= 1.003
    z = near + fmn * t

    for c in range(3):
        o_c = b[:, c * _MAX:(c + 1) * _MAX]
        d_c = b[:, (c + 3) * _MAX:(c + 4) * _MAX]
        pts_ref[c, :, :] = jnp.where(live, o_c + d_c * z, 0.0)

    h = jnp.maximum(
        jnp.dot(w1t_ref[...], ft, preferred_element_type=jnp.float32) + b1_ref[...],
        0.0)
    outt_ref[...] = jnp.dot(w2t_ref[...], h, preferred_element_type=jnp.float32) + b2_ref[...]


def kernel(rays_o, rays_d, bounds, distances, W1, b1, W2, b2):
    N = rays_o.shape[0]
    feats_t = jnp.concatenate([rays_o.T, rays_d.T, bounds.T, distances[None, :]], axis=0)
    R = _N_BLOCK
    grid = (N // R,)

    z_vals = _z_on_sparsecore(bounds.T, distances)

    pts_t, out_t = pl.pallas_call(
        _body,
        grid=grid,
        in_specs=[
            pl.BlockSpec((9, R), lambda i: (0, i)),
            pl.BlockSpec((9, 9 * _MAX), lambda i: (0, 0)),
            pl.BlockSpec((256, 9), lambda i: (0, 0)),
            pl.BlockSpec((256, 1), lambda i: (0, 0)),
            pl.BlockSpec((4, 256), lambda i: (0, 0)),
            pl.BlockSpec((4, 1), lambda i: (0, 0)),
        ],
        out_specs=[
            pl.BlockSpec((3, R, _MAX), lambda i: (0, i, 0)),
            pl.BlockSpec((4, R), lambda i: (0, i)),
        ],
        out_shape=[
            jax.ShapeDtypeStruct((3, N, _MAX), jnp.float32),
            jax.ShapeDtypeStruct((4, N), jnp.float32),
        ],
    )(feats_t, jnp.asarray(_SEL), W1.T, b1.reshape(256, 1), W2.T, b2.reshape(4, 1))

    return jnp.transpose(pts_t, (1, 2, 0)), z_vals, out_t.T
